# baseline (speedup 1.0000x reference)
import jax
import jax.numpy as jnp
import numpy as np
from jax.experimental import pallas as pl

INPUT_DIM = 768
VQ_DIM = 256
NUM_EMBED = 1024
SPLIT_NUM = 4
FREQ = INPUT_DIM // 2 + 1
FPAD = 512
KCH = 128

_k = np.arange(FPAD, dtype=np.int64)[:, None]
_i = np.arange(INPUT_DIM, dtype=np.int64)[None, :]
_ang = 2.0 * np.pi * ((_k * _i) % INPUT_DIM) / INPUT_DIM
_COS = np.cos(_ang).astype(np.float32)
_SIN = np.sin(_ang).astype(np.float32)
_COST = np.ascontiguousarray(_COS.T)
_SINT = np.ascontiguousarray(_SIN.T)
_HI = jax.lax.Precision.HIGHEST


def _tree_sum(parts):
    while len(parts) > 1:
        parts = [parts[i] + parts[i + 1] if i + 1 < len(parts) else parts[i]
                 for i in range(0, len(parts), 2)]
    return parts[0]


def _chunked_mm(a, b, precision):
    k = a.shape[1]
    parts = [jnp.dot(a[:, c:c + KCH], b[c:c + KCH, :], precision=precision)
             for c in range(0, k, KCH)]
    return _tree_sum(parts)


def _fold_kernel(cos_ref, sin_ref, cost_ref, sint_ref, rm_ref, b_ref):
    k = jax.lax.broadcasted_iota(jnp.int32, (FPAD, 1), 0)
    w = jnp.where((k == 0) | (k == INPUT_DIM // 2), 1.0, 2.0)
    rm = rm_ref[...]
    inv_n = 1.0 / INPUT_DIM
    c = cos_ref[...]
    s = sin_ref[...]
    ct = cost_ref[...]
    st = sint_ref[...]
    for sp in range(SPLIT_NUM):
        ds = jnp.where(rm == sp, w, 0.0)
        b = _chunked_mm(ct, ds * c, _HI) + _chunked_mm(st, ds * s, _HI)
        b_ref[:, sp * INPUT_DIM:(sp + 1) * INPUT_DIM] = b * inv_n


def _main_kernel(x_ref, b_ref, proj_ref, cbn_ref, out_ref):
    recon = _chunked_mm(x_ref[...], b_ref[...], _HI)
    proj = proj_ref[...]
    cbn = cbn_ref[...]
    lane = jax.lax.broadcasted_iota(
        jnp.int32, (x_ref.shape[0], NUM_EMBED), 1)
    cols = []
    for sp in range(SPLIT_NUM):
        feat = jnp.dot(recon[:, sp * INPUT_DIM:(sp + 1) * INPUT_DIM], proj)
        feat = feat / jnp.sqrt(jnp.sum(feat * feat, axis=1, keepdims=True))
        sim = jax.lax.dot_general(feat, cbn, (((1,), (1,)), ((), ())))
        m = jnp.max(sim, axis=1, keepdims=True)
        idx = jnp.min(jnp.where(sim == m, lane, NUM_EMBED),
                      axis=1, keepdims=True)
        cols.append(idx)
    out_ref[...] = jnp.concatenate(cols, axis=1).astype(jnp.int32)


def kernel(x, projector, codebook, random_matrix):
    b, t, n = x.shape
    rows = b * t
    xr = x.reshape(rows, n)
    rm = jnp.full((FPAD, 1), -1, dtype=jnp.int32)
    rm = rm.at[:FREQ, 0].set(random_matrix.astype(jnp.int32))
    cbn = (codebook / jnp.linalg.norm(codebook, axis=-1, keepdims=True)
           ).reshape(NUM_EMBED, VQ_DIM)

    bmat = pl.pallas_call(
        _fold_kernel,
        out_shape=jax.ShapeDtypeStruct((INPUT_DIM, SPLIT_NUM * INPUT_DIM),
                                       jnp.float32),
    )(jnp.asarray(_COS), jnp.asarray(_SIN), jnp.asarray(_COST),
      jnp.asarray(_SINT), rm)

    blk = 512
    out = pl.pallas_call(
        _main_kernel,
        grid=(rows // blk,),
        in_specs=[
            pl.BlockSpec((blk, n), lambda r: (r, 0)),
            pl.BlockSpec((INPUT_DIM, SPLIT_NUM * INPUT_DIM),
                         lambda r: (0, 0)),
            pl.BlockSpec((INPUT_DIM, VQ_DIM), lambda r: (0, 0)),
            pl.BlockSpec((NUM_EMBED, VQ_DIM), lambda r: (0, 0)),
        ],
        out_specs=pl.BlockSpec((blk, SPLIT_NUM), lambda r: (r, 0)),
        out_shape=jax.ShapeDtypeStruct((rows, SPLIT_NUM), jnp.int32),
    )(xr, bmat, projector, cbn)

    return out.reshape(b, t, SPLIT_NUM, 1)
